# Initial kernel scaffold; baseline (speedup 1.0000x reference)
import jax
import jax.numpy as jnp
from jax.experimental import pallas as pl


def kernel(x, edge_index, W1, b1, W2, b2, fcW, fcb):
    raise NotImplementedError("write your pallas kernel here")


# speedup vs baseline: 41.6956x; 41.6956x over previous
import functools

import jax
import jax.numpy as jnp
from jax import lax
from jax.experimental import pallas as pl
from jax.experimental.pallas import tpu as pltpu
from jax.experimental.pallas import tpu_sc as plsc

N = 10000
E = 640000
D_IN = 128
DH = 16

NC = 2
NS = 16
NW = NC * NS
B = 128
K = 8
CHUNK = B * K
EPW = 20480
E_PAD = NW * EPW
NCHUNK = EPW // CHUNK
ROWS2D = E_PAD // B
RPW = ROWS2D // NW
N_PAD = 10240
RPS = N_PAD // NS

_mesh = plsc.VectorSubcoreMesh(core_axis_name="c", subcore_axis_name="s")
_sc_params = pltpu.CompilerParams(use_tc_tiling_on_sc=False)


def _edge_scatter_body(src2d, dst2d, gtab, zblk, out, sidx, didx, rows, acc, sem):
    cid = lax.axis_index("c")
    sid = lax.axis_index("s")
    wid = sid * NC + cid

    pltpu.sync_copy(zblk, acc.at[pl.ds(sid * RPS, RPS)])
    plsc.subcore_barrier()

    def chunk(c, carry):
        rbase = wid * RPW + c * K
        pltpu.sync_copy(src2d.at[pl.ds(rbase, K)], sidx)
        pltpu.sync_copy(dst2d.at[pl.ds(rbase, K)], didx)
        handles = [
            pltpu.async_copy(gtab.at[sidx.at[j]], rows.at[j], sem)
            for j in range(K)
        ]
        for j in range(K):
            handles[j].wait()
        for j in range(K):
            pltpu.sync_copy(rows.at[j], acc.at[didx.at[j]], add=True)
        return carry

    lax.fori_loop(0, NCHUNK, chunk, 0)
    plsc.subcore_barrier()
    pltpu.sync_copy(
        acc.at[pl.ds(sid * RPS, RPS)], out.at[cid, pl.ds(sid * RPS, RPS)]
    )


def _deg_scatter_body(dst2d, oblk, zblk, out, didx, rows, acc, sem):
    cid = lax.axis_index("c")
    sid = lax.axis_index("s")
    wid = sid * NC + cid

    pltpu.sync_copy(zblk, acc.at[pl.ds(sid * RPS, RPS)])
    for j in range(K):
        pltpu.sync_copy(oblk, rows.at[j])
    plsc.subcore_barrier()

    def chunk(c, carry):
        rbase = wid * RPW + c * K
        pltpu.sync_copy(dst2d.at[pl.ds(rbase, K)], didx)
        for j in range(K):
            pltpu.sync_copy(rows.at[j], acc.at[didx.at[j]], add=True)
        return carry

    lax.fori_loop(0, NCHUNK, chunk, 0)
    plsc.subcore_barrier()
    pltpu.sync_copy(
        acc.at[pl.ds(sid * RPS, RPS)], out.at[cid, pl.ds(sid * RPS, RPS)]
    )
    del sem


_edge_scatter = functools.partial(
    pl.kernel,
    _edge_scatter_body,
    out_type=jax.ShapeDtypeStruct((NC, N_PAD, DH), jnp.float32),
    mesh=_mesh,
    scratch_types=[
        pltpu.VMEM((K, B), jnp.int32),
        pltpu.VMEM((K, B), jnp.int32),
        pltpu.VMEM((K, B, DH), jnp.float32),
        pltpu.VMEM_SHARED((N_PAD, DH), jnp.float32),
        pltpu.SemaphoreType.DMA,
    ],
    compiler_params=_sc_params,
)()

_deg_scatter = functools.partial(
    pl.kernel,
    _deg_scatter_body,
    out_type=jax.ShapeDtypeStruct((NC, N_PAD, DH), jnp.float32),
    mesh=_mesh,
    scratch_types=[
        pltpu.VMEM((K, B), jnp.int32),
        pltpu.VMEM((K, B, DH), jnp.float32),
        pltpu.VMEM_SHARED((N_PAD, DH), jnp.float32),
        pltpu.SemaphoreType.DMA,
    ],
    compiler_params=_sc_params,
)()


def _t1_body(x_ref, w1_ref, deg_ref, g1_ref, dinv_ref):
    deg = deg_ref[0][:, 0:1] + deg_ref[1][:, 0:1] + 1.0
    dinv = lax.rsqrt(deg)
    hw = jnp.dot(x_ref[...], w1_ref[...], preferred_element_type=jnp.float32)
    g1_ref[...] = hw * dinv
    dinv_ref[...] = dinv


def _t2_body(acc_ref, g1_ref, dinv_ref, b1_ref, w2_ref, g2_ref):
    dinv = dinv_ref[...]
    conv = (acc_ref[0] + acc_ref[1] + g1_ref[...]) * dinv
    h1 = jnp.maximum(conv + b1_ref[...], 0.0)
    rid = lax.broadcasted_iota(jnp.int32, (N_PAD, 1), 0)
    h1 = jnp.where(rid < N, h1, 0.0)
    g2_ref[...] = (
        jnp.dot(h1, w2_ref[...], preferred_element_type=jnp.float32) * dinv
    )


def _t3_body(acc_ref, g2_ref, dinv_ref, b2_ref, fcwt_ref, fcb_ref, out_ref):
    conv = (acc_ref[0] + acc_ref[1] + g2_ref[...]) * dinv_ref[...]
    h2 = jnp.maximum(conv + b2_ref[...], 0.0)
    rid = lax.broadcasted_iota(jnp.int32, (N_PAD, 1), 0)
    h2 = jnp.where(rid < N, h2, 0.0)
    pooled = jnp.sum(h2, axis=0, keepdims=True) * (1.0 / N)
    val = jnp.sum(pooled * fcwt_ref[...])
    out_ref[...] = val[None, None] + fcb_ref[...]


def kernel(x, edge_index, W1, b1, W2, b2, fcW, fcb):
    f32 = jnp.float32
    src = edge_index[0]
    dst = edge_index[1]
    pad = jnp.full((E_PAD - E,), N, jnp.int32)
    src2d = jnp.concatenate([src, pad]).reshape(ROWS2D, B)
    dst2d = jnp.concatenate([dst, pad]).reshape(ROWS2D, B)
    xp = jnp.pad(x, ((0, N_PAD - N), (0, 0)))
    zblk = jnp.zeros((RPS, DH), f32)
    oblk = jnp.ones((B, DH), f32)

    deg_acc = _deg_scatter(dst2d, oblk, zblk)

    g1, dinv = pl.pallas_call(
        _t1_body,
        out_shape=[
            jax.ShapeDtypeStruct((N_PAD, DH), f32),
            jax.ShapeDtypeStruct((N_PAD, 1), f32),
        ],
    )(xp, W1, deg_acc)

    acc1 = _edge_scatter(src2d, dst2d, g1, zblk)

    g2 = pl.pallas_call(
        _t2_body,
        out_shape=jax.ShapeDtypeStruct((N_PAD, DH), f32),
    )(acc1, g1, dinv, b1.reshape(1, DH), W2)

    acc2 = _edge_scatter(src2d, dst2d, g2, zblk)

    out = pl.pallas_call(
        _t3_body,
        out_shape=jax.ShapeDtypeStruct((1, 1), f32),
    )(acc2, g2, dinv, b2.reshape(1, DH), fcW.reshape(1, DH), fcb.reshape(1, 1))
    return out


# speedup vs baseline: 45.4657x; 1.0904x over previous
import functools

import jax
import jax.numpy as jnp
from jax import lax
from jax.experimental import pallas as pl
from jax.experimental.pallas import tpu as pltpu
from jax.experimental.pallas import tpu_sc as plsc

N = 10000
E = 640000
D_IN = 128
DH = 16

NC = 2
NS = 16
NW = NC * NS
B = 128
K = 8
CHUNK = B * K
EPW = 20480
E_PAD = NW * EPW
NCHUNK = EPW // CHUNK
ROWS2D = E_PAD // B
RPW = ROWS2D // NW
N_PAD = 10240
RPS = N_PAD // NS

_mesh = plsc.VectorSubcoreMesh(core_axis_name="c", subcore_axis_name="s")
_sc_params = pltpu.CompilerParams(use_tc_tiling_on_sc=False)


def _edge_scatter_body(src2d, dst2d, gtab, zblk, out, sidx, didx, rows, acc, sem, sem2):
    cid = lax.axis_index("c")
    sid = lax.axis_index("s")
    wid = sid * NC + cid

    pltpu.sync_copy(zblk, acc.at[pl.ds(sid * RPS, RPS)])
    plsc.subcore_barrier()

    def chunk(c, carry):
        rbase = wid * RPW + c * K
        pltpu.sync_copy(src2d.at[pl.ds(rbase, K)], sidx)
        pltpu.sync_copy(dst2d.at[pl.ds(rbase, K)], didx)
        gh = [
            pltpu.async_copy(gtab.at[sidx.at[j]], rows.at[j], sem)
            for j in range(K)
        ]
        sh = []
        for j in range(K):
            gh[j].wait()
            sh.append(
                pltpu.async_copy(rows.at[j], acc.at[didx.at[j]], sem2, add=True)
            )
        for j in range(K):
            sh[j].wait()
        return carry

    lax.fori_loop(0, NCHUNK, chunk, 0)
    plsc.subcore_barrier()
    pltpu.sync_copy(
        acc.at[pl.ds(sid * RPS, RPS)], out.at[cid, pl.ds(sid * RPS, RPS)]
    )


def _deg_scatter_body(dst2d, oblk, zblk, out, didx, rows, acc, sem):
    cid = lax.axis_index("c")
    sid = lax.axis_index("s")
    wid = sid * NC + cid

    pltpu.sync_copy(zblk, acc.at[pl.ds(sid * RPS, RPS)])
    for j in range(K):
        pltpu.sync_copy(oblk, rows.at[j])
    plsc.subcore_barrier()

    def chunk(c, carry):
        rbase = wid * RPW + c * K
        pltpu.sync_copy(dst2d.at[pl.ds(rbase, K)], didx)
        sh = [
            pltpu.async_copy(rows.at[j], acc.at[didx.at[j]], sem, add=True)
            for j in range(K)
        ]
        for j in range(K):
            sh[j].wait()
        return carry

    lax.fori_loop(0, NCHUNK, chunk, 0)
    plsc.subcore_barrier()
    pltpu.sync_copy(
        acc.at[pl.ds(sid * RPS, RPS)], out.at[cid, pl.ds(sid * RPS, RPS)]
    )


_edge_scatter = functools.partial(
    pl.kernel,
    _edge_scatter_body,
    out_type=jax.ShapeDtypeStruct((NC, N_PAD, DH), jnp.float32),
    mesh=_mesh,
    scratch_types=[
        pltpu.VMEM((K, B), jnp.int32),
        pltpu.VMEM((K, B), jnp.int32),
        pltpu.VMEM((K, B, DH), jnp.float32),
        pltpu.VMEM_SHARED((N_PAD, DH), jnp.float32),
        pltpu.SemaphoreType.DMA,
        pltpu.SemaphoreType.DMA,
    ],
    compiler_params=_sc_params,
)()

_deg_scatter = functools.partial(
    pl.kernel,
    _deg_scatter_body,
    out_type=jax.ShapeDtypeStruct((NC, N_PAD, DH), jnp.float32),
    mesh=_mesh,
    scratch_types=[
        pltpu.VMEM((K, B), jnp.int32),
        pltpu.VMEM((K, B, DH), jnp.float32),
        pltpu.VMEM_SHARED((N_PAD, DH), jnp.float32),
        pltpu.SemaphoreType.DMA,
    ],
    compiler_params=_sc_params,
)()


def _t1_body(x_ref, w1_ref, deg_ref, g1_ref, dinv_ref):
    deg = deg_ref[0][:, 0:1] + deg_ref[1][:, 0:1] + 1.0
    dinv = lax.rsqrt(deg)
    hw = jnp.dot(x_ref[...], w1_ref[...], preferred_element_type=jnp.float32)
    g1_ref[...] = hw * dinv
    dinv_ref[...] = dinv


def _t2_body(acc_ref, g1_ref, dinv_ref, b1_ref, w2_ref, g2_ref):
    dinv = dinv_ref[...]
    conv = (acc_ref[0] + acc_ref[1] + g1_ref[...]) * dinv
    h1 = jnp.maximum(conv + b1_ref[...], 0.0)
    rid = lax.broadcasted_iota(jnp.int32, (N_PAD, 1), 0)
    h1 = jnp.where(rid < N, h1, 0.0)
    g2_ref[...] = (
        jnp.dot(h1, w2_ref[...], preferred_element_type=jnp.float32) * dinv
    )


def _t3_body(acc_ref, g2_ref, dinv_ref, b2_ref, fcwt_ref, fcb_ref, out_ref):
    conv = (acc_ref[0] + acc_ref[1] + g2_ref[...]) * dinv_ref[...]
    h2 = jnp.maximum(conv + b2_ref[...], 0.0)
    rid = lax.broadcasted_iota(jnp.int32, (N_PAD, 1), 0)
    h2 = jnp.where(rid < N, h2, 0.0)
    pooled = jnp.sum(h2, axis=0, keepdims=True) * (1.0 / N)
    val = jnp.sum(pooled * fcwt_ref[...])
    out_ref[...] = val[None, None] + fcb_ref[...]


def kernel(x, edge_index, W1, b1, W2, b2, fcW, fcb):
    f32 = jnp.float32
    src = edge_index[0]
    dst = edge_index[1]
    pad = jnp.full((E_PAD - E,), N, jnp.int32)
    src2d = jnp.concatenate([src, pad]).reshape(ROWS2D, B)
    dst2d = jnp.concatenate([dst, pad]).reshape(ROWS2D, B)
    xp = jnp.pad(x, ((0, N_PAD - N), (0, 0)))
    zblk = jnp.zeros((RPS, DH), f32)
    oblk = jnp.ones((B, DH), f32)

    deg_acc = _deg_scatter(dst2d, oblk, zblk)

    g1, dinv = pl.pallas_call(
        _t1_body,
        out_shape=[
            jax.ShapeDtypeStruct((N_PAD, DH), f32),
            jax.ShapeDtypeStruct((N_PAD, 1), f32),
        ],
    )(xp, W1, deg_acc)

    acc1 = _edge_scatter(src2d, dst2d, g1, zblk)

    g2 = pl.pallas_call(
        _t2_body,
        out_shape=jax.ShapeDtypeStruct((N_PAD, DH), f32),
    )(acc1, g1, dinv, b1.reshape(1, DH), W2)

    acc2 = _edge_scatter(src2d, dst2d, g2, zblk)

    out = pl.pallas_call(
        _t3_body,
        out_shape=jax.ShapeDtypeStruct((1, 1), f32),
    )(acc2, g2, dinv, b2.reshape(1, DH), fcW.reshape(1, DH), fcb.reshape(1, 1))
    return out


# speedup vs baseline: 52.5934x; 1.1568x over previous
import functools

import jax
import jax.numpy as jnp
from jax import lax
from jax.experimental import pallas as pl
from jax.experimental.pallas import tpu as pltpu
from jax.experimental.pallas import tpu_sc as plsc

N = 10000
E = 640000
D_IN = 128
DH = 16

NC = 2
NS = 16
NW = NC * NS
B = 128
K = 16
EPW = 20480
E_PAD = NW * EPW
ROWS2D = E_PAD // B
RPW = ROWS2D // NW
NBODY = RPW // K
N_PAD = 10240
RPS = N_PAD // NS

_mesh = plsc.VectorSubcoreMesh(core_axis_name="c", subcore_axis_name="s")
_sc_params = pltpu.CompilerParams(use_tc_tiling_on_sc=False)


def _edge_scatter_body(src2d, dst2d, gtab, zblk, out, sidx, didx, rows, acc,
                       gsem, ssem):
    cid = lax.axis_index("c")
    sid = lax.axis_index("s")
    wid = sid * NC + cid

    pltpu.sync_copy(zblk, acc.at[pl.ds(sid * RPS, RPS)])
    pltpu.sync_copy(src2d.at[pl.ds(wid * RPW, RPW)], sidx)
    pltpu.sync_copy(dst2d.at[pl.ds(wid * RPW, RPW)], didx)
    plsc.subcore_barrier()

    def body(i, carry):
        r0 = i * K
        gh = [
            pltpu.async_copy(gtab.at[sidx.at[r0 + j]], rows.at[j], gsem)
            for j in range(K)
        ]
        sh = []
        for j in range(K):
            gh[j].wait()
            sh.append(
                pltpu.async_copy(rows.at[j], acc.at[didx.at[r0 + j]], ssem,
                                 add=True)
            )
        for h in sh:
            h.wait()
        return carry

    lax.fori_loop(0, NBODY, body, 0)
    plsc.subcore_barrier()
    pltpu.sync_copy(
        acc.at[pl.ds(sid * RPS, RPS)], out.at[cid, pl.ds(sid * RPS, RPS)]
    )


def _deg_scatter_body(dst2d, oblk, zblk, out, didx, rows, acc, ssem):
    cid = lax.axis_index("c")
    sid = lax.axis_index("s")
    wid = sid * NC + cid

    pltpu.sync_copy(zblk, acc.at[pl.ds(sid * RPS, RPS)])
    pltpu.sync_copy(dst2d.at[pl.ds(wid * RPW, RPW)], didx)
    pltpu.sync_copy(oblk, rows)
    plsc.subcore_barrier()

    def body(i, carry):
        r0 = i * K
        sh = [
            pltpu.async_copy(rows, acc.at[didx.at[r0 + j]], ssem, add=True)
            for j in range(K)
        ]
        for h in sh:
            h.wait()
        return carry

    lax.fori_loop(0, NBODY, body, 0)
    plsc.subcore_barrier()
    pltpu.sync_copy(
        acc.at[pl.ds(sid * RPS, RPS)], out.at[cid, pl.ds(sid * RPS, RPS)]
    )


_edge_scatter = functools.partial(
    pl.kernel,
    _edge_scatter_body,
    out_type=jax.ShapeDtypeStruct((NC, N_PAD, DH), jnp.float32),
    mesh=_mesh,
    scratch_types=[
        pltpu.VMEM((RPW, B), jnp.int32),
        pltpu.VMEM((RPW, B), jnp.int32),
        pltpu.VMEM((K, B, DH), jnp.float32),
        pltpu.VMEM_SHARED((N_PAD, DH), jnp.float32),
        pltpu.SemaphoreType.DMA,
        pltpu.SemaphoreType.DMA,
    ],
    compiler_params=_sc_params,
)()

_deg_scatter = functools.partial(
    pl.kernel,
    _deg_scatter_body,
    out_type=jax.ShapeDtypeStruct((NC, N_PAD, DH), jnp.float32),
    mesh=_mesh,
    scratch_types=[
        pltpu.VMEM((RPW, B), jnp.int32),
        pltpu.VMEM((B, DH), jnp.float32),
        pltpu.VMEM_SHARED((N_PAD, DH), jnp.float32),
        pltpu.SemaphoreType.DMA,
    ],
    compiler_params=_sc_params,
)()


def _t1_body(x_ref, w1_ref, deg_ref, g1_ref, dinv_ref):
    deg = deg_ref[0][:, 0:1] + deg_ref[1][:, 0:1] + 1.0
    dinv = lax.rsqrt(deg)
    hw = jnp.dot(x_ref[...], w1_ref[...], preferred_element_type=jnp.float32)
    g1_ref[...] = hw * dinv
    dinv_ref[...] = dinv


def _t2_body(acc_ref, g1_ref, dinv_ref, b1_ref, w2_ref, g2_ref):
    dinv = dinv_ref[...]
    conv = (acc_ref[0] + acc_ref[1] + g1_ref[...]) * dinv
    h1 = jnp.maximum(conv + b1_ref[...], 0.0)
    rid = lax.broadcasted_iota(jnp.int32, (N_PAD, 1), 0)
    h1 = jnp.where(rid < N, h1, 0.0)
    g2_ref[...] = (
        jnp.dot(h1, w2_ref[...], preferred_element_type=jnp.float32) * dinv
    )


def _t3_body(acc_ref, g2_ref, dinv_ref, b2_ref, fcwt_ref, fcb_ref, out_ref):
    conv = (acc_ref[0] + acc_ref[1] + g2_ref[...]) * dinv_ref[...]
    h2 = jnp.maximum(conv + b2_ref[...], 0.0)
    rid = lax.broadcasted_iota(jnp.int32, (N_PAD, 1), 0)
    h2 = jnp.where(rid < N, h2, 0.0)
    pooled = jnp.sum(h2, axis=0, keepdims=True) * (1.0 / N)
    val = jnp.sum(pooled * fcwt_ref[...])
    out_ref[...] = val[None, None] + fcb_ref[...]


def kernel(x, edge_index, W1, b1, W2, b2, fcW, fcb):
    f32 = jnp.float32
    src = edge_index[0]
    dst = edge_index[1]
    pad = jnp.full((E_PAD - E,), N, jnp.int32)
    src2d = jnp.concatenate([src, pad]).reshape(ROWS2D, B)
    dst2d = jnp.concatenate([dst, pad]).reshape(ROWS2D, B)
    xp = jnp.pad(x, ((0, N_PAD - N), (0, 0)))
    zblk = jnp.zeros((RPS, DH), f32)
    oblk = jnp.ones((B, DH), f32)

    deg_acc = _deg_scatter(dst2d, oblk, zblk)

    g1, dinv = pl.pallas_call(
        _t1_body,
        out_shape=[
            jax.ShapeDtypeStruct((N_PAD, DH), f32),
            jax.ShapeDtypeStruct((N_PAD, 1), f32),
        ],
    )(xp, W1, deg_acc)

    acc1 = _edge_scatter(src2d, dst2d, g1, zblk)

    g2 = pl.pallas_call(
        _t2_body,
        out_shape=jax.ShapeDtypeStruct((N_PAD, DH), f32),
    )(acc1, g1, dinv, b1.reshape(1, DH), W2)

    acc2 = _edge_scatter(src2d, dst2d, g2, zblk)

    out = pl.pallas_call(
        _t3_body,
        out_shape=jax.ShapeDtypeStruct((1, 1), f32),
    )(acc2, g2, dinv, b2.reshape(1, DH), fcW.reshape(1, DH), fcb.reshape(1, 1))
    return out
